# baseline (speedup 1.0000x reference)
import functools
import jax
import jax.numpy as jnp
from jax import lax
from jax.experimental import pallas as pl
from jax.experimental.pallas import tpu as pltpu
from jax.experimental.pallas import tpu_sc as plsc

N_INSTRUMENTS = 1000
REVERB_LENGTH = 24000
BATCH = 1024

NC, NS, L = 2, 16, 16
NW = NC * NS
CHUNKS = 8
DC = REVERB_LENGTH // CHUNKS
B_PER_W = BATCH // NW
PAIRS = B_PER_W // 2


def _body(idx_hbm, table_hbm, out_hbm, idx_v, idxc_v, buf, sem):
    wid = lax.axis_index("s") * NC + lax.axis_index("c")
    base = wid * B_PER_W

    pltpu.sync_copy(idx_hbm.at[pl.ds(base * 1, B_PER_W)], idx_v)

    iota = lax.iota(jnp.int32, L)
    lane_chunk = iota & (CHUNKS - 1)
    lane_row = iota >> 3

    for h in range(B_PER_W // L):
        idx16 = idx_v[pl.ds(h * L, L)]
        for q in range(L // 2):
            p = h * (L // 2) + q
            rows = idx16.at[2 * q + lane_row].get(mode="promise_in_bounds")
            vec = rows * CHUNKS + lane_chunk
            idxc_v[...] = vec
            pltpu.async_copy(table_hbm.at[idxc_v], buf, sem).wait()
            pltpu.sync_copy(buf, out_hbm.at[pl.ds((base + 2 * p) * CHUNKS, L)])


@jax.jit
def _gather(idx, table2):
    mesh = plsc.VectorSubcoreMesh(core_axis_name="c", subcore_axis_name="s")
    run = pl.kernel(
        _body,
        out_type=jax.ShapeDtypeStruct((BATCH * CHUNKS, DC), jnp.float32),
        mesh=mesh,
        scratch_types=[
            pltpu.VMEM((B_PER_W,), jnp.int32),
            pltpu.VMEM((L,), jnp.int32),
            pltpu.VMEM((L, DC), jnp.float32),
            pltpu.SemaphoreType.DMA,
        ],
        compiler_params=pltpu.CompilerParams(use_tc_tiling_on_sc=False),
    )
    return run(idx, table2)


def kernel(piano_model, reverb_dict_weight):
    idx = piano_model.astype(jnp.int32)
    table2 = reverb_dict_weight.reshape(N_INSTRUMENTS * CHUNKS, DC)
    out2 = _gather(idx, table2)
    return out2.reshape(BATCH, REVERB_LENGTH)


# speedup vs baseline: 1.0270x; 1.0270x over previous
import jax
import jax.numpy as jnp
from jax import lax
from jax.experimental import pallas as pl
from jax.experimental.pallas import tpu as pltpu
from jax.experimental.pallas import tpu_sc as plsc

N_INSTRUMENTS = 1000
REVERB_LENGTH = 24000
BATCH = 1024

NC, NS, L = 2, 16, 16
NW = NC * NS
CHUNKS = 8
DC = REVERB_LENGTH // CHUNKS
B_PER_W = BATCH // NW
PAIRS = B_PER_W // 2
NBUF = 2


def _body(idx_hbm, table_hbm, out_hbm, idx_v, idxc_v, bufs, gsems, wsems):
    idxc_v = list(idxc_v)
    bufs = list(bufs)
    gsems = list(gsems)
    wsems = list(wsems)

    wid = lax.axis_index("s") * NC + lax.axis_index("c")
    base = wid * B_PER_W

    pltpu.sync_copy(idx_hbm.at[pl.ds(base, B_PER_W)], idx_v)

    iota = lax.iota(jnp.int32, L)
    lane_chunk = iota & (CHUNKS - 1)
    lane_row = iota >> 3

    def build_idx(p, slot):
        idx16 = idx_v[pl.ds((2 * p // L) * L, L)]
        rows = idx16.at[(2 * p) % L + lane_row].get(mode="promise_in_bounds")
        idxc_v[slot][...] = rows * CHUNKS + lane_chunk

    def start_gather(slot):
        pltpu.async_copy(table_hbm.at[idxc_v[slot]], bufs[slot], gsems[slot])

    def write_copy(p, slot):
        return pltpu.make_async_copy(
            bufs[slot], out_hbm.at[pl.ds((base + 2 * p) * CHUNKS, L)],
            wsems[slot])

    for s in range(NBUF):
        build_idx(s, s)
        start_gather(s)

    for p in range(PAIRS):
        s = p % NBUF
        pltpu.make_async_copy(table_hbm.at[idxc_v[s]], bufs[s],
                              gsems[s]).wait()
        write_copy(p, s).start()
        if p + NBUF < PAIRS:
            build_idx(p + NBUF, s)
            write_copy(p, s).wait()
            start_gather(s)

    for p in range(PAIRS - NBUF, PAIRS):
        write_copy(p, p % NBUF).wait()


@jax.jit
def _gather(idx, table2):
    mesh = plsc.VectorSubcoreMesh(core_axis_name="c", subcore_axis_name="s")
    run = pl.kernel(
        _body,
        out_type=jax.ShapeDtypeStruct((BATCH * CHUNKS, DC), jnp.float32),
        mesh=mesh,
        scratch_types=[
            pltpu.VMEM((B_PER_W,), jnp.int32),
            [pltpu.VMEM((L,), jnp.int32) for _ in range(NBUF)],
            [pltpu.VMEM((L, DC), jnp.float32) for _ in range(NBUF)],
            [pltpu.SemaphoreType.DMA for _ in range(NBUF)],
            [pltpu.SemaphoreType.DMA for _ in range(NBUF)],
        ],
        compiler_params=pltpu.CompilerParams(use_tc_tiling_on_sc=False),
    )
    return run(idx, table2)


def kernel(piano_model, reverb_dict_weight):
    idx = piano_model.astype(jnp.int32)
    table2 = reverb_dict_weight.reshape(N_INSTRUMENTS * CHUNKS, DC)
    out2 = _gather(idx, table2)
    return out2.reshape(BATCH, REVERB_LENGTH)
